# baseline (speedup 1.0000x reference)
import functools

import jax
import jax.numpy as jnp
from jax import lax
from jax.experimental import pallas as pl
from jax.experimental.pallas import tpu as pltpu
from jax.experimental.pallas import tpu_sc as plsc

H = 2048
I = 1024
NE = 8
TOPK = 2
NGRP = 4
TOPKG = 2
N = 2048
TILE = 256
PADDED = N * TOPK + NE * TILE
NUM_TILES = PADDED // TILE
NWORK = 32
TOK_PER_W = N // NWORK
SUB = 16
NSUB = TOK_PER_W // SUB


def _routing_body(x_ref, gw_ref, eb_ref, pos_a_ref, pos_b_ref, w_a_ref,
                  w_b_ref, eot_ref, act_ref, cum_ref, ch_ref):
    f32 = jnp.float32
    logits = lax.dot_general(gw_ref[...], x_ref[...],
                             (((1,), (1,)), ((), ())),
                             preferred_element_type=f32)
    scores = 1.0 / (1.0 + jnp.exp(-logits))
    sfc = scores + eb_ref[...]
    s = [sfc[e:e + 1, :] for e in range(NE)]
    sc = [scores[e:e + 1, :] for e in range(NE)]
    g = [s[2 * j] + s[2 * j + 1] for j in range(NGRP)]
    gmask = []
    for j in range(NGRP):
        rank = jnp.zeros_like(g[0])
        for k in range(NGRP):
            gt = jnp.where(g[k] > g[j], 1.0, 0.0)
            eqlt = jnp.where(g[k] == g[j], 1.0, 0.0) if k < j else 0.0
            rank = rank + gt + eqlt
        gmask.append(jnp.where(rank < TOPKG, 1.0, 0.0))
    tmp = [s[e] * gmask[e // 2] for e in range(NE)]
    ch = []
    for e in range(NE):
        rank = jnp.zeros_like(tmp[0])
        for f in range(NE):
            gt = jnp.where(tmp[f] > tmp[e], 1.0, 0.0)
            eqlt = jnp.where(tmp[f] == tmp[e], 1.0, 0.0) if f < e else 0.0
            rank = rank + gt + eqlt
        ch.append(jnp.where(rank < TOPK, 1.0, 0.0))
    wraw = [sc[e] * ch[e] for e in range(NE)]
    denom = wraw[0]
    for e in range(1, NE):
        denom = denom + wraw[e]
    denom = denom + 1e-20
    wn = [wraw[e] / denom for e in range(NE)]

    ch_ref[...] = jnp.concatenate(ch, axis=0)
    r0 = lax.broadcasted_iota(jnp.int32, (128, 128), 0)
    c0 = lax.broadcasted_iota(jnp.int32, (128, 128), 1)
    tri = jnp.where(r0 < c0, 1.0, 0.0).astype(f32)

    def chunk(i, carry):
        blk = ch_ref[:, pl.ds(i * 128, 128)]
        cum = lax.dot_general(blk, tri, (((1,), (0,)), ((), ())),
                              preferred_element_type=f32) + carry
        cum_ref[:, pl.ds(i * 128, 128)] = cum
        return carry + jnp.sum(blk, axis=1, keepdims=True)

    cnt = lax.fori_loop(0, N // 128, chunk, jnp.zeros((NE, 1), f32))
    pci = (cnt.astype(jnp.int32) + (TILE - 1)) & (-TILE)
    pcf = pci.astype(f32)
    r8 = lax.broadcasted_iota(jnp.int32, (NE, NE), 0)
    c8 = lax.broadcasted_iota(jnp.int32, (NE, NE), 1)
    tri8 = jnp.where(c8 < r8, 1.0, 0.0).astype(f32)
    off = lax.dot_general(tri8, pcf, (((1,), (0,)), ((), ())),
                          preferred_element_type=f32)
    total = jnp.sum(pcf)

    posfull = cum_ref[...] + off
    run = jnp.zeros_like(ch[0])
    pos_a = jnp.zeros_like(ch[0])
    pos_b = jnp.zeros_like(ch[0])
    w_a = jnp.zeros_like(ch[0])
    w_b = jnp.zeros_like(ch[0])
    for e in range(NE):
        ia = ch[e] * jnp.where(run == 0.0, 1.0, 0.0)
        ib = ch[e] * jnp.where(run == 1.0, 1.0, 0.0)
        pe = posfull[e:e + 1, :]
        pos_a = pos_a + ia * pe
        pos_b = pos_b + ib * pe
        w_a = w_a + ia * wn[e]
        w_b = w_b + ib * wn[e]
        run = run + ch[e]
    pos_a_ref[...] = pos_a.astype(jnp.int32)
    pos_b_ref[...] = pos_b.astype(jnp.int32)
    w_a_ref[...] = w_a
    w_b_ref[...] = w_b

    mt = (lax.broadcasted_iota(jnp.int32, (NE, 128), 1) * TILE).astype(f32)
    cmp = jnp.where(mt >= off, 1.0, 0.0)
    eot = jnp.sum(cmp, axis=0, keepdims=True) - 1.0
    eot_ref[...] = eot.astype(jnp.int32)
    mtr = (lax.broadcasted_iota(jnp.int32, (1, 128), 1) * TILE).astype(f32)
    act_ref[...] = jnp.where(mtr < total, 1, 0).astype(jnp.int32)


def _routing_call(x, gate_weight, e_bias):
    out_shapes = (
        jax.ShapeDtypeStruct((1, N), jnp.int32),
        jax.ShapeDtypeStruct((1, N), jnp.int32),
        jax.ShapeDtypeStruct((1, N), jnp.float32),
        jax.ShapeDtypeStruct((1, N), jnp.float32),
        jax.ShapeDtypeStruct((1, 128), jnp.int32),
        jax.ShapeDtypeStruct((1, 128), jnp.int32),
    )
    return pl.pallas_call(
        _routing_body,
        out_shape=out_shapes,
        scratch_shapes=[pltpu.VMEM((NE, N), jnp.float32),
                        pltpu.VMEM((NE, N), jnp.float32)],
    )(x, gate_weight, e_bias)


def _scatter_body(x_hbm, ia_hbm, ib_hbm, xsort_hbm, ia_v, ib_v, xbuf):
    wid = lax.axis_index("s") * 2 + lax.axis_index("c")
    pltpu.sync_copy(ia_hbm.at[wid], ia_v)
    pltpu.sync_copy(ib_hbm.at[wid], ib_v)
    for j in range(NSUB):
        base = wid * TOK_PER_W + j * SUB
        pltpu.sync_copy(x_hbm.at[pl.ds(base, SUB)], xbuf)
        pltpu.sync_copy(xbuf, xsort_hbm.at[ia_v.at[j]])
        pltpu.sync_copy(xbuf, xsort_hbm.at[ib_v.at[j]])


def _scatter_call(x, ia, ib):
    mesh = plsc.VectorSubcoreMesh(core_axis_name="c", subcore_axis_name="s")
    kern = pl.kernel(
        _scatter_body,
        out_type=jax.ShapeDtypeStruct((PADDED, H), jnp.float32),
        mesh=mesh,
        scratch_types=[
            pltpu.VMEM((NSUB, SUB), jnp.int32),
            pltpu.VMEM((NSUB, SUB), jnp.int32),
            pltpu.VMEM((SUB, H), jnp.float32),
        ],
    )
    return kern(x, ia, ib)


def _gateup_body(eot_ref, act_ref, xs_ref, g_ref, u_ref, h_ref, gbf_ref,
                 ubf_ref):
    m = pl.program_id(0)
    bf16 = jnp.bfloat16
    new_e = jnp.logical_or(m == 0, eot_ref[m] != eot_ref[jnp.maximum(m - 1, 0)])

    @pl.when(jnp.logical_and(act_ref[m] == 1, new_e))
    def _():
        gbf_ref[...] = g_ref[0].astype(bf16)
        ubf_ref[...] = u_ref[0].astype(bf16)

    @pl.when(act_ref[m] == 1)
    def _():
        xb = xs_ref[...].astype(bf16)
        gg = lax.dot_general(xb, gbf_ref[...], (((1,), (1,)), ((), ())),
                             preferred_element_type=jnp.float32)
        uu = lax.dot_general(xb, ubf_ref[...], (((1,), (1,)), ((), ())),
                             preferred_element_type=jnp.float32)
        h_ref[...] = (gg / (1.0 + jnp.exp(-gg)) * uu).astype(bf16)


def _down_body(eot_ref, act_ref, h_ref, d_ref, o_ref, dbf_ref):
    m = pl.program_id(0)
    new_e = jnp.logical_or(m == 0, eot_ref[m] != eot_ref[jnp.maximum(m - 1, 0)])

    @pl.when(jnp.logical_and(act_ref[m] == 1, new_e))
    def _():
        dbf_ref[...] = d_ref[0].astype(jnp.bfloat16)

    @pl.when(act_ref[m] == 1)
    def _():
        o_ref[...] = lax.dot_general(h_ref[...], dbf_ref[...],
                                     (((1,), (1,)), ((), ())),
                                     preferred_element_type=jnp.float32)


def _ffn_call(eot, act, xsort, gate_proj, up_proj, down_proj):
    gu_spec = pltpu.PrefetchScalarGridSpec(
        num_scalar_prefetch=2,
        grid=(NUM_TILES,),
        in_specs=[
            pl.BlockSpec((TILE, H), lambda m, eot, act: (m, 0)),
            pl.BlockSpec((1, I, H), lambda m, eot, act: (eot[m], 0, 0)),
            pl.BlockSpec((1, I, H), lambda m, eot, act: (eot[m], 0, 0)),
        ],
        out_specs=pl.BlockSpec((TILE, I), lambda m, eot, act: (m, 0)),
        scratch_shapes=[pltpu.VMEM((I, H), jnp.bfloat16),
                        pltpu.VMEM((I, H), jnp.bfloat16)],
    )
    hsort = pl.pallas_call(
        _gateup_body,
        grid_spec=gu_spec,
        out_shape=jax.ShapeDtypeStruct((PADDED, I), jnp.bfloat16),
        compiler_params=pltpu.CompilerParams(
            dimension_semantics=("arbitrary",)),
    )(eot, act, xsort, gate_proj, up_proj)

    dn_spec = pltpu.PrefetchScalarGridSpec(
        num_scalar_prefetch=2,
        grid=(NUM_TILES,),
        in_specs=[
            pl.BlockSpec((TILE, I), lambda m, eot, act: (m, 0)),
            pl.BlockSpec((1, H, I), lambda m, eot, act: (eot[m], 0, 0)),
        ],
        out_specs=pl.BlockSpec((TILE, H), lambda m, eot, act: (m, 0)),
        scratch_shapes=[pltpu.VMEM((H, I), jnp.bfloat16)],
    )
    return pl.pallas_call(
        _down_body,
        grid_spec=dn_spec,
        out_shape=jax.ShapeDtypeStruct((PADDED, H), jnp.float32),
        compiler_params=pltpu.CompilerParams(
            dimension_semantics=("arbitrary",)),
    )(eot, act, hsort, down_proj)


def _gather_body(os_hbm, ia_hbm, ib_hbm, ga_hbm, gb_hbm, ia_v, ib_v, abuf,
                 bbuf):
    wid = lax.axis_index("s") * 2 + lax.axis_index("c")
    pltpu.sync_copy(ia_hbm.at[wid], ia_v)
    pltpu.sync_copy(ib_hbm.at[wid], ib_v)
    for j in range(NSUB):
        base = wid * TOK_PER_W + j * SUB
        pltpu.sync_copy(os_hbm.at[ia_v.at[j]], abuf)
        pltpu.sync_copy(os_hbm.at[ib_v.at[j]], bbuf)
        pltpu.sync_copy(abuf, ga_hbm.at[pl.ds(base, SUB)])
        pltpu.sync_copy(bbuf, gb_hbm.at[pl.ds(base, SUB)])


def _gather_call(outsort, ia, ib):
    mesh = plsc.VectorSubcoreMesh(core_axis_name="c", subcore_axis_name="s")
    kern = pl.kernel(
        _gather_body,
        out_type=(jax.ShapeDtypeStruct((N, H), jnp.float32),
                  jax.ShapeDtypeStruct((N, H), jnp.float32)),
        mesh=mesh,
        scratch_types=[
            pltpu.VMEM((NSUB, SUB), jnp.int32),
            pltpu.VMEM((NSUB, SUB), jnp.int32),
            pltpu.VMEM((SUB, H), jnp.float32),
            pltpu.VMEM((SUB, H), jnp.float32),
        ],
    )
    return kern(outsort, ia, ib)


def _combine_body(ga_ref, gb_ref, wa_ref, wb_ref, y_ref):
    y_ref[...] = wa_ref[...] * ga_ref[...] + wb_ref[...] * gb_ref[...]


def _combine_call(ga, gb, wa, wb):
    return pl.pallas_call(
        _combine_body,
        grid=(N // TILE,),
        in_specs=[
            pl.BlockSpec((TILE, H), lambda m: (m, 0)),
            pl.BlockSpec((TILE, H), lambda m: (m, 0)),
            pl.BlockSpec((TILE, 1), lambda m: (m, 0)),
            pl.BlockSpec((TILE, 1), lambda m: (m, 0)),
        ],
        out_specs=pl.BlockSpec((TILE, H), lambda m: (m, 0)),
        out_shape=jax.ShapeDtypeStruct((N, H), jnp.float32),
    )(ga, gb, wa, wb)


def kernel(hidden_states, gate_weight, e_bias, gate_proj, up_proj, down_proj):
    b, sq, h = hidden_states.shape
    x = hidden_states.reshape(N, H).astype(jnp.float32)
    eb = e_bias.reshape(NE, 1).astype(jnp.float32)

    pos_a, pos_b, w_a, w_b, eot, act = _routing_call(x, gate_weight, eb)
    ia_sc = pos_a.reshape(NWORK, NSUB, SUB)
    ib_sc = pos_b.reshape(NWORK, NSUB, SUB)

    xsort = _scatter_call(x, ia_sc, ib_sc)
    outsort = _ffn_call(eot[0, :NUM_TILES], act[0, :NUM_TILES], xsort,
                        gate_proj, up_proj, down_proj)
    ga, gb = _gather_call(outsort, ia_sc, ib_sc)
    y = _combine_call(ga, gb, w_a.reshape(N, 1), w_b.reshape(N, 1))
    return y.reshape(b, sq, h)


# speedup vs baseline: 1.1418x; 1.1418x over previous
import functools

import jax
import jax.numpy as jnp
from jax import lax
from jax.experimental import pallas as pl
from jax.experimental.pallas import tpu as pltpu
from jax.experimental.pallas import tpu_sc as plsc

H = 2048
I = 1024
NE = 8
TOPK = 2
NGRP = 4
TOPKG = 2
N = 2048
TILE = 256
PADDED = N * TOPK + NE * TILE
NUM_TILES = PADDED // TILE
NWORK = 32
TOK_PER_W = N // NWORK
SUB = 16
NSUB = TOK_PER_W // SUB


def _routing_body(x_ref, gw_ref, eb_ref, pos_a_ref, pos_b_ref, w_a_ref,
                  w_b_ref, eot_ref, act_ref, cum_ref, ch_ref):
    f32 = jnp.float32
    logits = lax.dot_general(gw_ref[...], x_ref[...],
                             (((1,), (1,)), ((), ())),
                             preferred_element_type=f32)
    scores = 1.0 / (1.0 + jnp.exp(-logits))
    sfc = scores + eb_ref[...]
    s = [sfc[e:e + 1, :] for e in range(NE)]
    sc = [scores[e:e + 1, :] for e in range(NE)]
    g = [s[2 * j] + s[2 * j + 1] for j in range(NGRP)]
    gmask = []
    for j in range(NGRP):
        rank = jnp.zeros_like(g[0])
        for k in range(NGRP):
            gt = jnp.where(g[k] > g[j], 1.0, 0.0)
            eqlt = jnp.where(g[k] == g[j], 1.0, 0.0) if k < j else 0.0
            rank = rank + gt + eqlt
        gmask.append(jnp.where(rank < TOPKG, 1.0, 0.0))
    tmp = [s[e] * gmask[e // 2] for e in range(NE)]
    ch = []
    for e in range(NE):
        rank = jnp.zeros_like(tmp[0])
        for f in range(NE):
            gt = jnp.where(tmp[f] > tmp[e], 1.0, 0.0)
            eqlt = jnp.where(tmp[f] == tmp[e], 1.0, 0.0) if f < e else 0.0
            rank = rank + gt + eqlt
        ch.append(jnp.where(rank < TOPK, 1.0, 0.0))
    wraw = [sc[e] * ch[e] for e in range(NE)]
    denom = wraw[0]
    for e in range(1, NE):
        denom = denom + wraw[e]
    denom = denom + 1e-20
    wn = [wraw[e] / denom for e in range(NE)]

    ch_ref[...] = jnp.concatenate(ch, axis=0)
    r0 = lax.broadcasted_iota(jnp.int32, (128, 128), 0)
    c0 = lax.broadcasted_iota(jnp.int32, (128, 128), 1)
    tri = jnp.where(r0 < c0, 1.0, 0.0).astype(f32)

    def chunk(i, carry):
        blk = ch_ref[:, pl.ds(i * 128, 128)]
        cum = lax.dot_general(blk, tri, (((1,), (0,)), ((), ())),
                              preferred_element_type=f32) + carry
        cum_ref[:, pl.ds(i * 128, 128)] = cum
        return carry + jnp.sum(blk, axis=1, keepdims=True)

    cnt = lax.fori_loop(0, N // 128, chunk, jnp.zeros((NE, 1), f32))
    pci = (cnt.astype(jnp.int32) + (TILE - 1)) & (-TILE)
    pcf = pci.astype(f32)
    r8 = lax.broadcasted_iota(jnp.int32, (NE, NE), 0)
    c8 = lax.broadcasted_iota(jnp.int32, (NE, NE), 1)
    tri8 = jnp.where(c8 < r8, 1.0, 0.0).astype(f32)
    off = lax.dot_general(tri8, pcf, (((1,), (0,)), ((), ())),
                          preferred_element_type=f32)
    total = jnp.sum(pcf)

    posfull = cum_ref[...] + off
    run = jnp.zeros_like(ch[0])
    pos_a = jnp.zeros_like(ch[0])
    pos_b = jnp.zeros_like(ch[0])
    w_a = jnp.zeros_like(ch[0])
    w_b = jnp.zeros_like(ch[0])
    for e in range(NE):
        ia = ch[e] * jnp.where(run == 0.0, 1.0, 0.0)
        ib = ch[e] * jnp.where(run == 1.0, 1.0, 0.0)
        pe = posfull[e:e + 1, :]
        pos_a = pos_a + ia * pe
        pos_b = pos_b + ib * pe
        w_a = w_a + ia * wn[e]
        w_b = w_b + ib * wn[e]
        run = run + ch[e]
    pos_a_ref[...] = pos_a.astype(jnp.int32)
    pos_b_ref[...] = pos_b.astype(jnp.int32)
    w_a_ref[...] = w_a
    w_b_ref[...] = w_b

    mt = (lax.broadcasted_iota(jnp.int32, (NE, 128), 1) * TILE).astype(f32)
    cmp = jnp.where(mt >= off, 1.0, 0.0)
    eot = jnp.sum(cmp, axis=0, keepdims=True) - 1.0
    eot_ref[...] = eot.astype(jnp.int32)
    mtr = (lax.broadcasted_iota(jnp.int32, (1, 128), 1) * TILE).astype(f32)
    act_ref[...] = jnp.where(mtr < total, 1, 0).astype(jnp.int32)


def _routing_call(x, gate_weight, e_bias):
    out_shapes = (
        jax.ShapeDtypeStruct((1, N), jnp.int32),
        jax.ShapeDtypeStruct((1, N), jnp.int32),
        jax.ShapeDtypeStruct((1, N), jnp.float32),
        jax.ShapeDtypeStruct((1, N), jnp.float32),
        jax.ShapeDtypeStruct((1, 128), jnp.int32),
        jax.ShapeDtypeStruct((1, 128), jnp.int32),
    )
    return pl.pallas_call(
        _routing_body,
        out_shape=out_shapes,
        scratch_shapes=[pltpu.VMEM((NE, N), jnp.float32),
                        pltpu.VMEM((NE, N), jnp.float32)],
    )(x, gate_weight, e_bias)


def _scatter_body(x_hbm, ia_hbm, ib_hbm, xsort_hbm, ia_v, ib_v, xbuf):
    wid = lax.axis_index("s") * 2 + lax.axis_index("c")
    pltpu.sync_copy(ia_hbm.at[wid], ia_v)
    pltpu.sync_copy(ib_hbm.at[wid], ib_v)
    for j in range(NSUB):
        base = wid * TOK_PER_W + j * SUB
        pltpu.sync_copy(x_hbm.at[pl.ds(base, SUB)], xbuf)
        pltpu.sync_copy(xbuf, xsort_hbm.at[ia_v.at[j]])
        pltpu.sync_copy(xbuf, xsort_hbm.at[ib_v.at[j]])


def _scatter_call(x, ia, ib):
    mesh = plsc.VectorSubcoreMesh(core_axis_name="c", subcore_axis_name="s")
    kern = pl.kernel(
        _scatter_body,
        out_type=jax.ShapeDtypeStruct((PADDED, H), jnp.float32),
        mesh=mesh,
        scratch_types=[
            pltpu.VMEM((NSUB, SUB), jnp.int32),
            pltpu.VMEM((NSUB, SUB), jnp.int32),
            pltpu.VMEM((SUB, H), jnp.float32),
        ],
    )
    return kern(x, ia, ib)


_DOT_ALG = lax.Precision.DEFAULT


def _ffn_body(eot_ref, act_ref, xs_ref, g_ref, u_ref, d_ref, o_ref):
    @pl.when(act_ref[pl.program_id(0)] == 1)
    def _():
        f32 = jnp.float32
        xt = xs_ref[...]
        gg = lax.dot_general(xt, g_ref[0], (((1,), (1,)), ((), ())),
                             precision=_DOT_ALG, preferred_element_type=f32)
        uu = lax.dot_general(xt, u_ref[0], (((1,), (1,)), ((), ())),
                             precision=_DOT_ALG, preferred_element_type=f32)
        hh = gg / (1.0 + jnp.exp(-gg)) * uu
        o_ref[...] = lax.dot_general(hh, d_ref[0], (((1,), (1,)), ((), ())),
                                     precision=_DOT_ALG,
                                     preferred_element_type=f32)


def _ffn_call(eot, act, xsort, gate_proj, up_proj, down_proj):
    grid_spec = pltpu.PrefetchScalarGridSpec(
        num_scalar_prefetch=2,
        grid=(NUM_TILES,),
        in_specs=[
            pl.BlockSpec((TILE, H), lambda m, eot, act: (m, 0)),
            pl.BlockSpec((1, I, H), lambda m, eot, act: (eot[m], 0, 0)),
            pl.BlockSpec((1, I, H), lambda m, eot, act: (eot[m], 0, 0)),
            pl.BlockSpec((1, H, I), lambda m, eot, act: (eot[m], 0, 0)),
        ],
        out_specs=pl.BlockSpec((TILE, H), lambda m, eot, act: (m, 0)),
    )
    return pl.pallas_call(
        _ffn_body,
        grid_spec=grid_spec,
        out_shape=jax.ShapeDtypeStruct((PADDED, H), jnp.float32),
        compiler_params=pltpu.CompilerParams(
            dimension_semantics=("arbitrary",)),
    )(eot, act, xsort, gate_proj, up_proj, down_proj)


def _gather_body(os_hbm, ia_hbm, ib_hbm, ga_hbm, gb_hbm, ia_v, ib_v, abuf,
                 bbuf):
    wid = lax.axis_index("s") * 2 + lax.axis_index("c")
    pltpu.sync_copy(ia_hbm.at[wid], ia_v)
    pltpu.sync_copy(ib_hbm.at[wid], ib_v)
    for j in range(NSUB):
        base = wid * TOK_PER_W + j * SUB
        pltpu.sync_copy(os_hbm.at[ia_v.at[j]], abuf)
        pltpu.sync_copy(os_hbm.at[ib_v.at[j]], bbuf)
        pltpu.sync_copy(abuf, ga_hbm.at[pl.ds(base, SUB)])
        pltpu.sync_copy(bbuf, gb_hbm.at[pl.ds(base, SUB)])


def _gather_call(outsort, ia, ib):
    mesh = plsc.VectorSubcoreMesh(core_axis_name="c", subcore_axis_name="s")
    kern = pl.kernel(
        _gather_body,
        out_type=(jax.ShapeDtypeStruct((N, H), jnp.float32),
                  jax.ShapeDtypeStruct((N, H), jnp.float32)),
        mesh=mesh,
        scratch_types=[
            pltpu.VMEM((NSUB, SUB), jnp.int32),
            pltpu.VMEM((NSUB, SUB), jnp.int32),
            pltpu.VMEM((SUB, H), jnp.float32),
            pltpu.VMEM((SUB, H), jnp.float32),
        ],
    )
    return kern(outsort, ia, ib)


def _combine_body(ga_ref, gb_ref, wa_ref, wb_ref, y_ref):
    y_ref[...] = wa_ref[...] * ga_ref[...] + wb_ref[...] * gb_ref[...]


def _combine_call(ga, gb, wa, wb):
    return pl.pallas_call(
        _combine_body,
        grid=(N // TILE,),
        in_specs=[
            pl.BlockSpec((TILE, H), lambda m: (m, 0)),
            pl.BlockSpec((TILE, H), lambda m: (m, 0)),
            pl.BlockSpec((TILE, 1), lambda m: (m, 0)),
            pl.BlockSpec((TILE, 1), lambda m: (m, 0)),
        ],
        out_specs=pl.BlockSpec((TILE, H), lambda m: (m, 0)),
        out_shape=jax.ShapeDtypeStruct((N, H), jnp.float32),
    )(ga, gb, wa, wb)


def kernel(hidden_states, gate_weight, e_bias, gate_proj, up_proj, down_proj):
    b, sq, h = hidden_states.shape
    x = hidden_states.reshape(N, H).astype(jnp.float32)
    eb = e_bias.reshape(NE, 1).astype(jnp.float32)

    pos_a, pos_b, w_a, w_b, eot, act = _routing_call(x, gate_weight, eb)
    ia_sc = pos_a.reshape(NWORK, NSUB, SUB)
    ib_sc = pos_b.reshape(NWORK, NSUB, SUB)

    xsort = _scatter_call(x, ia_sc, ib_sc)
    outsort = _ffn_call(eot[0, :NUM_TILES], act[0, :NUM_TILES], xsort,
                        gate_proj, up_proj, down_proj)
    ga, gb = _gather_call(outsort, ia_sc, ib_sc)
    y = _combine_call(ga, gb, w_a.reshape(N, 1), w_b.reshape(N, 1))
    return y.reshape(b, sq, h)


# speedup vs baseline: 1.2467x; 1.0919x over previous
import functools

import jax
import jax.numpy as jnp
from jax import lax
from jax.experimental import pallas as pl
from jax.experimental.pallas import tpu as pltpu
from jax.experimental.pallas import tpu_sc as plsc

H = 2048
I = 1024
NE = 8
TOPK = 2
NGRP = 4
TOPKG = 2
N = 2048
TILE = 256
PADDED = N * TOPK + NE * TILE
NUM_TILES = PADDED // TILE
NWORK = 32
TOK_PER_W = N // NWORK
SUB = 16
NSUB = TOK_PER_W // SUB


def _pack_rows(v):
    u = lax.bitcast_convert_type(v, jnp.uint32)
    c2 = v.shape[1] // 2
    ah, bh = u[:, :c2], u[:, c2:]
    one, s16 = jnp.uint32(1), jnp.uint32(16)
    rnd = jnp.uint32(0x7FFF)
    ar = (ah + rnd + ((ah >> s16) & one)) & jnp.uint32(0xFFFF0000)
    br = (bh + rnd + ((bh >> s16) & one)) >> s16
    return lax.bitcast_convert_type(ar | br, jnp.int32)


def _unpack_rows(p):
    u = lax.bitcast_convert_type(p, jnp.uint32)
    hi = lax.bitcast_convert_type(u & jnp.uint32(0xFFFF0000), jnp.float32)
    lo = lax.bitcast_convert_type(u << jnp.uint32(16), jnp.float32)
    return jnp.concatenate([hi, lo], axis=1)


def _routing_body(x_ref, gw_ref, eb_ref, pos_a_ref, pos_b_ref, w_a_ref,
                  w_b_ref, eot_ref, act_ref, xpk_ref, cum_ref, ch_ref):
    f32 = jnp.float32
    xpk_ref[...] = _pack_rows(x_ref[...])
    logits = lax.dot_general(gw_ref[...], x_ref[...],
                             (((1,), (1,)), ((), ())),
                             preferred_element_type=f32)
    scores = 1.0 / (1.0 + jnp.exp(-logits))
    sfc = scores + eb_ref[...]
    s = [sfc[e:e + 1, :] for e in range(NE)]
    sc = [scores[e:e + 1, :] for e in range(NE)]
    g = [s[2 * j] + s[2 * j + 1] for j in range(NGRP)]
    gmask = []
    for j in range(NGRP):
        rank = jnp.zeros_like(g[0])
        for k in range(NGRP):
            gt = jnp.where(g[k] > g[j], 1.0, 0.0)
            eqlt = jnp.where(g[k] == g[j], 1.0, 0.0) if k < j else 0.0
            rank = rank + gt + eqlt
        gmask.append(jnp.where(rank < TOPKG, 1.0, 0.0))
    tmp = [s[e] * gmask[e // 2] for e in range(NE)]
    ch = []
    for e in range(NE):
        rank = jnp.zeros_like(tmp[0])
        for f in range(NE):
            gt = jnp.where(tmp[f] > tmp[e], 1.0, 0.0)
            eqlt = jnp.where(tmp[f] == tmp[e], 1.0, 0.0) if f < e else 0.0
            rank = rank + gt + eqlt
        ch.append(jnp.where(rank < TOPK, 1.0, 0.0))
    wraw = [sc[e] * ch[e] for e in range(NE)]
    denom = wraw[0]
    for e in range(1, NE):
        denom = denom + wraw[e]
    denom = denom + 1e-20
    wn = [wraw[e] / denom for e in range(NE)]

    ch_ref[...] = jnp.concatenate(ch, axis=0)
    r0 = lax.broadcasted_iota(jnp.int32, (128, 128), 0)
    c0 = lax.broadcasted_iota(jnp.int32, (128, 128), 1)
    tri = jnp.where(r0 < c0, 1.0, 0.0).astype(f32)

    def chunk(i, carry):
        blk = ch_ref[:, pl.ds(i * 128, 128)]
        cum = lax.dot_general(blk, tri, (((1,), (0,)), ((), ())),
                              preferred_element_type=f32) + carry
        cum_ref[:, pl.ds(i * 128, 128)] = cum
        return carry + jnp.sum(blk, axis=1, keepdims=True)

    cnt = lax.fori_loop(0, N // 128, chunk, jnp.zeros((NE, 1), f32))
    pci = (cnt.astype(jnp.int32) + (TILE - 1)) & (-TILE)
    pcf = pci.astype(f32)
    r8 = lax.broadcasted_iota(jnp.int32, (NE, NE), 0)
    c8 = lax.broadcasted_iota(jnp.int32, (NE, NE), 1)
    tri8 = jnp.where(c8 < r8, 1.0, 0.0).astype(f32)
    off = lax.dot_general(tri8, pcf, (((1,), (0,)), ((), ())),
                          preferred_element_type=f32)
    total = jnp.sum(pcf)

    posfull = cum_ref[...] + off
    run = jnp.zeros_like(ch[0])
    pos_a = jnp.zeros_like(ch[0])
    pos_b = jnp.zeros_like(ch[0])
    w_a = jnp.zeros_like(ch[0])
    w_b = jnp.zeros_like(ch[0])
    for e in range(NE):
        ia = ch[e] * jnp.where(run == 0.0, 1.0, 0.0)
        ib = ch[e] * jnp.where(run == 1.0, 1.0, 0.0)
        pe = posfull[e:e + 1, :]
        pos_a = pos_a + ia * pe
        pos_b = pos_b + ib * pe
        w_a = w_a + ia * wn[e]
        w_b = w_b + ib * wn[e]
        run = run + ch[e]
    pos_a_ref[...] = pos_a.astype(jnp.int32)
    pos_b_ref[...] = pos_b.astype(jnp.int32)
    w_a_ref[...] = w_a
    w_b_ref[...] = w_b

    mt = (lax.broadcasted_iota(jnp.int32, (NE, 128), 1) * TILE).astype(f32)
    cmp = jnp.where(mt >= off, 1.0, 0.0)
    eot = jnp.sum(cmp, axis=0, keepdims=True) - 1.0
    eot_ref[...] = eot.astype(jnp.int32)
    mtr = (lax.broadcasted_iota(jnp.int32, (1, 128), 1) * TILE).astype(f32)
    act_ref[...] = jnp.where(mtr < total, 1, 0).astype(jnp.int32)


def _routing_call(x, gate_weight, e_bias):
    out_shapes = (
        jax.ShapeDtypeStruct((1, N), jnp.int32),
        jax.ShapeDtypeStruct((1, N), jnp.int32),
        jax.ShapeDtypeStruct((1, N), jnp.float32),
        jax.ShapeDtypeStruct((1, N), jnp.float32),
        jax.ShapeDtypeStruct((1, 128), jnp.int32),
        jax.ShapeDtypeStruct((1, 128), jnp.int32),
        jax.ShapeDtypeStruct((N, H // 2), jnp.int32),
    )
    return pl.pallas_call(
        _routing_body,
        out_shape=out_shapes,
        scratch_shapes=[pltpu.VMEM((NE, N), jnp.float32),
                        pltpu.VMEM((NE, N), jnp.float32)],
    )(x, gate_weight, e_bias)


def _scatter_body(x_hbm, ia_hbm, ib_hbm, xsort_hbm, ia_v, ib_v, xbuf):
    wid = lax.axis_index("s") * 2 + lax.axis_index("c")
    pltpu.sync_copy(ia_hbm.at[wid], ia_v)
    pltpu.sync_copy(ib_hbm.at[wid], ib_v)
    for j in range(NSUB):
        base = wid * TOK_PER_W + j * SUB
        pltpu.sync_copy(x_hbm.at[pl.ds(base, SUB)], xbuf)
        pltpu.sync_copy(xbuf, xsort_hbm.at[ia_v.at[j]])
        pltpu.sync_copy(xbuf, xsort_hbm.at[ib_v.at[j]])


def _scatter_call(x, ia, ib):
    mesh = plsc.VectorSubcoreMesh(core_axis_name="c", subcore_axis_name="s")
    kern = pl.kernel(
        _scatter_body,
        out_type=jax.ShapeDtypeStruct((PADDED, H // 2), jnp.int32),
        mesh=mesh,
        scratch_types=[
            pltpu.VMEM((NSUB, SUB), jnp.int32),
            pltpu.VMEM((NSUB, SUB), jnp.int32),
            pltpu.VMEM((SUB, H // 2), jnp.int32),
        ],
    )
    return kern(x, ia, ib)


_DOT_ALG = lax.Precision.DEFAULT


def _ffn_body(eot_ref, act_ref, xs_ref, g_ref, u_ref, d_ref, o_ref):
    @pl.when(act_ref[pl.program_id(0)] == 1)
    def _():
        f32 = jnp.float32
        xt = _unpack_rows(xs_ref[...])
        gg = lax.dot_general(xt, g_ref[0], (((1,), (1,)), ((), ())),
                             precision=_DOT_ALG, preferred_element_type=f32)
        uu = lax.dot_general(xt, u_ref[0], (((1,), (1,)), ((), ())),
                             precision=_DOT_ALG, preferred_element_type=f32)
        hh = gg / (1.0 + jnp.exp(-gg)) * uu
        oo = lax.dot_general(hh, d_ref[0], (((1,), (1,)), ((), ())),
                             precision=_DOT_ALG, preferred_element_type=f32)
        o_ref[...] = _pack_rows(oo)


def _ffn_call(eot, act, xsort, gate_proj, up_proj, down_proj):
    grid_spec = pltpu.PrefetchScalarGridSpec(
        num_scalar_prefetch=2,
        grid=(NUM_TILES,),
        in_specs=[
            pl.BlockSpec((TILE, H // 2), lambda m, eot, act: (m, 0)),
            pl.BlockSpec((1, I, H), lambda m, eot, act: (eot[m], 0, 0)),
            pl.BlockSpec((1, I, H), lambda m, eot, act: (eot[m], 0, 0)),
            pl.BlockSpec((1, H, I), lambda m, eot, act: (eot[m], 0, 0)),
        ],
        out_specs=pl.BlockSpec((TILE, H // 2), lambda m, eot, act: (m, 0)),
    )
    return pl.pallas_call(
        _ffn_body,
        grid_spec=grid_spec,
        out_shape=jax.ShapeDtypeStruct((PADDED, H // 2), jnp.int32),
        compiler_params=pltpu.CompilerParams(
            dimension_semantics=("arbitrary",)),
    )(eot, act, xsort, gate_proj, up_proj, down_proj)


def _gather_body(os_hbm, ia_hbm, ib_hbm, ga_hbm, gb_hbm, ia_v, ib_v, abuf,
                 bbuf):
    wid = lax.axis_index("s") * 2 + lax.axis_index("c")
    pltpu.sync_copy(ia_hbm.at[wid], ia_v)
    pltpu.sync_copy(ib_hbm.at[wid], ib_v)
    for j in range(NSUB):
        base = wid * TOK_PER_W + j * SUB
        pltpu.sync_copy(os_hbm.at[ia_v.at[j]], abuf)
        pltpu.sync_copy(os_hbm.at[ib_v.at[j]], bbuf)
        pltpu.sync_copy(abuf, ga_hbm.at[pl.ds(base, SUB)])
        pltpu.sync_copy(bbuf, gb_hbm.at[pl.ds(base, SUB)])


def _gather_call(outsort, ia, ib):
    mesh = plsc.VectorSubcoreMesh(core_axis_name="c", subcore_axis_name="s")
    kern = pl.kernel(
        _gather_body,
        out_type=(jax.ShapeDtypeStruct((N, H // 2), jnp.int32),
                  jax.ShapeDtypeStruct((N, H // 2), jnp.int32)),
        mesh=mesh,
        scratch_types=[
            pltpu.VMEM((NSUB, SUB), jnp.int32),
            pltpu.VMEM((NSUB, SUB), jnp.int32),
            pltpu.VMEM((SUB, H // 2), jnp.int32),
            pltpu.VMEM((SUB, H // 2), jnp.int32),
        ],
    )
    return kern(outsort, ia, ib)


def _combine_body(ga_ref, gb_ref, wa_ref, wb_ref, y_ref):
    y_ref[...] = (wa_ref[...] * _unpack_rows(ga_ref[...])
                  + wb_ref[...] * _unpack_rows(gb_ref[...]))


def _combine_call(ga, gb, wa, wb):
    return pl.pallas_call(
        _combine_body,
        grid=(N // TILE,),
        in_specs=[
            pl.BlockSpec((TILE, H // 2), lambda m: (m, 0)),
            pl.BlockSpec((TILE, H // 2), lambda m: (m, 0)),
            pl.BlockSpec((TILE, 1), lambda m: (m, 0)),
            pl.BlockSpec((TILE, 1), lambda m: (m, 0)),
        ],
        out_specs=pl.BlockSpec((TILE, H), lambda m: (m, 0)),
        out_shape=jax.ShapeDtypeStruct((N, H), jnp.float32),
    )(ga, gb, wa, wb)


def kernel(hidden_states, gate_weight, e_bias, gate_proj, up_proj, down_proj):
    b, sq, h = hidden_states.shape
    x = hidden_states.reshape(N, H).astype(jnp.float32)
    eb = e_bias.reshape(NE, 1).astype(jnp.float32)

    pos_a, pos_b, w_a, w_b, eot, act, xpk = _routing_call(x, gate_weight, eb)
    ia_sc = pos_a.reshape(NWORK, NSUB, SUB)
    ib_sc = pos_b.reshape(NWORK, NSUB, SUB)

    xsort = _scatter_call(xpk, ia_sc, ib_sc)
    outsort = _ffn_call(eot[0, :NUM_TILES], act[0, :NUM_TILES], xsort,
                        gate_proj, up_proj, down_proj)
    ga, gb = _gather_call(outsort, ia_sc, ib_sc)
    y = _combine_call(ga, gb, w_a.reshape(N, 1), w_b.reshape(N, 1))
    return y.reshape(b, sq, h)


# speedup vs baseline: 1.4160x; 1.1358x over previous
import functools

import jax
import jax.numpy as jnp
from jax import lax
from jax.experimental import pallas as pl
from jax.experimental.pallas import tpu as pltpu
from jax.experimental.pallas import tpu_sc as plsc

H = 2048
I = 1024
NE = 8
TOPK = 2
NGRP = 4
TOPKG = 2
N = 2048
TILE = 256
PADDED = N * TOPK + NE * TILE
NUM_TILES = PADDED // TILE
NWORK = 32
TOK_PER_W = N // NWORK
SUB = 16
NSUB = TOK_PER_W // SUB


def _pack_rows(v):
    u = lax.bitcast_convert_type(v, jnp.uint32)
    c2 = v.shape[1] // 2
    ah, bh = u[:, :c2], u[:, c2:]
    one, s16 = jnp.uint32(1), jnp.uint32(16)
    rnd = jnp.uint32(0x7FFF)
    ar = (ah + rnd + ((ah >> s16) & one)) & jnp.uint32(0xFFFF0000)
    br = (bh + rnd + ((bh >> s16) & one)) >> s16
    return lax.bitcast_convert_type(ar | br, jnp.int32)


def _unpack_rows(p):
    u = lax.bitcast_convert_type(p, jnp.uint32)
    hi = lax.bitcast_convert_type(u & jnp.uint32(0xFFFF0000), jnp.float32)
    lo = lax.bitcast_convert_type(u << jnp.uint32(16), jnp.float32)
    return jnp.concatenate([hi, lo], axis=1)


def _routing_body(x_ref, gw_ref, eb_ref, pos_a_ref, pos_b_ref, w_a_ref,
                  w_b_ref, eot_ref, act_ref, par_ref, nxt_ref, xpk_ref,
                  cum_ref, ch_ref):
    f32 = jnp.float32
    xpk_ref[...] = _pack_rows(x_ref[...])
    logits = lax.dot_general(gw_ref[...], x_ref[...],
                             (((1,), (1,)), ((), ())),
                             preferred_element_type=f32)
    scores = 1.0 / (1.0 + jnp.exp(-logits))
    sfc = scores + eb_ref[...]
    s = [sfc[e:e + 1, :] for e in range(NE)]
    sc = [scores[e:e + 1, :] for e in range(NE)]
    g = [s[2 * j] + s[2 * j + 1] for j in range(NGRP)]
    gmask = []
    for j in range(NGRP):
        rank = jnp.zeros_like(g[0])
        for k in range(NGRP):
            gt = jnp.where(g[k] > g[j], 1.0, 0.0)
            eqlt = jnp.where(g[k] == g[j], 1.0, 0.0) if k < j else 0.0
            rank = rank + gt + eqlt
        gmask.append(jnp.where(rank < TOPKG, 1.0, 0.0))
    tmp = [s[e] * gmask[e // 2] for e in range(NE)]
    ch = []
    for e in range(NE):
        rank = jnp.zeros_like(tmp[0])
        for f in range(NE):
            gt = jnp.where(tmp[f] > tmp[e], 1.0, 0.0)
            eqlt = jnp.where(tmp[f] == tmp[e], 1.0, 0.0) if f < e else 0.0
            rank = rank + gt + eqlt
        ch.append(jnp.where(rank < TOPK, 1.0, 0.0))
    wraw = [sc[e] * ch[e] for e in range(NE)]
    denom = wraw[0]
    for e in range(1, NE):
        denom = denom + wraw[e]
    denom = denom + 1e-20
    wn = [wraw[e] / denom for e in range(NE)]

    ch_ref[...] = jnp.concatenate(ch, axis=0)
    r0 = lax.broadcasted_iota(jnp.int32, (128, 128), 0)
    c0 = lax.broadcasted_iota(jnp.int32, (128, 128), 1)
    tri = jnp.where(r0 < c0, 1.0, 0.0).astype(f32)

    def chunk(i, carry):
        blk = ch_ref[:, pl.ds(i * 128, 128)]
        cum = lax.dot_general(blk, tri, (((1,), (0,)), ((), ())),
                              preferred_element_type=f32) + carry
        cum_ref[:, pl.ds(i * 128, 128)] = cum
        return carry + jnp.sum(blk, axis=1, keepdims=True)

    cnt = lax.fori_loop(0, N // 128, chunk, jnp.zeros((NE, 1), f32))
    pci = (cnt.astype(jnp.int32) + (TILE - 1)) & (-TILE)
    pcf = pci.astype(f32)
    r8 = lax.broadcasted_iota(jnp.int32, (NE, NE), 0)
    c8 = lax.broadcasted_iota(jnp.int32, (NE, NE), 1)
    tri8 = jnp.where(c8 < r8, 1.0, 0.0).astype(f32)
    off = lax.dot_general(tri8, pcf, (((1,), (0,)), ((), ())),
                          preferred_element_type=f32)
    total = jnp.sum(pcf)

    posfull = cum_ref[...] + off
    run = jnp.zeros_like(ch[0])
    pos_a = jnp.zeros_like(ch[0])
    pos_b = jnp.zeros_like(ch[0])
    w_a = jnp.zeros_like(ch[0])
    w_b = jnp.zeros_like(ch[0])
    for e in range(NE):
        ia = ch[e] * jnp.where(run == 0.0, 1.0, 0.0)
        ib = ch[e] * jnp.where(run == 1.0, 1.0, 0.0)
        pe = posfull[e:e + 1, :]
        pos_a = pos_a + ia * pe
        pos_b = pos_b + ib * pe
        w_a = w_a + ia * wn[e]
        w_b = w_b + ib * wn[e]
        run = run + ch[e]
    pos_a_ref[...] = pos_a.astype(jnp.int32)
    pos_b_ref[...] = pos_b.astype(jnp.int32)
    w_a_ref[...] = w_a
    w_b_ref[...] = w_b

    mt = (lax.broadcasted_iota(jnp.int32, (NE, 128), 1) * TILE).astype(f32)
    cmp = jnp.where(mt >= off, 1.0, 0.0)
    eot = jnp.sum(cmp, axis=0, keepdims=True) - 1.0
    eot_ref[...] = eot.astype(jnp.int32)
    mtr = (lax.broadcasted_iota(jnp.int32, (1, 128), 1) * TILE).astype(f32)
    act_ref[...] = jnp.where(mtr < total, 1, 0).astype(jnp.int32)

    end = off + pcf
    seg_done = jnp.where((mt >= end) & (pcf > 0.0), 1.0, 0.0)
    segord = jnp.sum(seg_done, axis=0, keepdims=True)
    par_ref[...] = (segord.astype(jnp.int32)) & 1
    intile = jnp.where((mt >= off) & (mt < end), 1.0, 0.0)
    end_tile = jnp.sum(intile * end, axis=0, keepdims=True)
    ind = jnp.where((off <= end_tile) & (end_tile < end), 1.0, 0.0)
    eidx = lax.broadcasted_iota(jnp.int32, (NE, 128), 0).astype(f32)
    nxt = (jnp.sum(ind * eidx, axis=0, keepdims=True)
           + jnp.sum(ind, axis=0, keepdims=True) - 1.0)
    nxt_ref[...] = nxt.astype(jnp.int32)


def _routing_call(x, gate_weight, e_bias):
    out_shapes = (
        jax.ShapeDtypeStruct((1, N), jnp.int32),
        jax.ShapeDtypeStruct((1, N), jnp.int32),
        jax.ShapeDtypeStruct((1, N), jnp.float32),
        jax.ShapeDtypeStruct((1, N), jnp.float32),
        jax.ShapeDtypeStruct((1, 128), jnp.int32),
        jax.ShapeDtypeStruct((1, 128), jnp.int32),
        jax.ShapeDtypeStruct((1, 128), jnp.int32),
        jax.ShapeDtypeStruct((1, 128), jnp.int32),
        jax.ShapeDtypeStruct((N, H // 2), jnp.int32),
    )
    return pl.pallas_call(
        _routing_body,
        out_shape=out_shapes,
        scratch_shapes=[pltpu.VMEM((NE, N), jnp.float32),
                        pltpu.VMEM((NE, N), jnp.float32)],
    )(x, gate_weight, e_bias)


def _scatter_body(x_hbm, ia_hbm, ib_hbm, xsort_hbm, ia_v, ib_v, xbuf):
    wid = lax.axis_index("s") * 2 + lax.axis_index("c")
    pltpu.sync_copy(ia_hbm.at[wid], ia_v)
    pltpu.sync_copy(ib_hbm.at[wid], ib_v)
    for j in range(NSUB):
        base = wid * TOK_PER_W + j * SUB
        pltpu.sync_copy(x_hbm.at[pl.ds(base, SUB)], xbuf)
        pltpu.sync_copy(xbuf, xsort_hbm.at[ia_v.at[j]])
        pltpu.sync_copy(xbuf, xsort_hbm.at[ib_v.at[j]])


def _scatter_call(x, ia, ib):
    mesh = plsc.VectorSubcoreMesh(core_axis_name="c", subcore_axis_name="s")
    kern = pl.kernel(
        _scatter_body,
        out_type=jax.ShapeDtypeStruct((PADDED, H // 2), jnp.int32),
        mesh=mesh,
        scratch_types=[
            pltpu.VMEM((NSUB, SUB), jnp.int32),
            pltpu.VMEM((NSUB, SUB), jnp.int32),
            pltpu.VMEM((SUB, H // 2), jnp.int32),
        ],
    )
    return kern(x, ia, ib)


_DOT_ALG = lax.Precision.DEFAULT


def _issue_weights(g_ref, u_ref, d_ref, gbuf, ubuf, dbuf, sems, e, slot):
    pltpu.make_async_copy(g_ref.at[e], gbuf.at[slot], sems.at[slot]).start()
    pltpu.make_async_copy(u_ref.at[e], ubuf.at[slot], sems.at[slot]).start()
    pltpu.make_async_copy(d_ref.at[e], dbuf.at[slot], sems.at[slot]).start()


def _wait_weights(g_ref, u_ref, d_ref, gbuf, ubuf, dbuf, sems, e, slot):
    pltpu.make_async_copy(g_ref.at[e], gbuf.at[slot], sems.at[slot]).wait()
    pltpu.make_async_copy(u_ref.at[e], ubuf.at[slot], sems.at[slot]).wait()
    pltpu.make_async_copy(d_ref.at[e], dbuf.at[slot], sems.at[slot]).wait()


def _ffn_body(eot_ref, act_ref, par_ref, nxt_ref, xs_ref, g_ref, u_ref, d_ref,
              o_ref, gbuf, ubuf, dbuf, sems):
    m = pl.program_id(0)
    e = eot_ref[m]
    p = par_ref[m]
    active = act_ref[m] == 1
    new_e = jnp.logical_or(m == 0, e != eot_ref[jnp.maximum(m - 1, 0)])

    @pl.when(m == 0)
    def _():
        _issue_weights(g_ref, u_ref, d_ref, gbuf, ubuf, dbuf, sems, e, 0)

    @pl.when(jnp.logical_and(active, new_e))
    def _():
        _wait_weights(g_ref, u_ref, d_ref, gbuf, ubuf, dbuf, sems, e, p)
        nx = nxt_ref[m]

        @pl.when(nx >= 0)
        def _():
            _issue_weights(g_ref, u_ref, d_ref, gbuf, ubuf, dbuf, sems, nx,
                           1 - p)

    @pl.when(active)
    def _():
        f32 = jnp.float32
        xt = _unpack_rows(xs_ref[...])
        gw = gbuf[pl.ds(p, 1)][0]
        uw = ubuf[pl.ds(p, 1)][0]
        dw = dbuf[pl.ds(p, 1)][0]
        gg = lax.dot_general(xt, gw, (((1,), (1,)), ((), ())),
                             precision=_DOT_ALG, preferred_element_type=f32)
        uu = lax.dot_general(xt, uw, (((1,), (1,)), ((), ())),
                             precision=_DOT_ALG, preferred_element_type=f32)
        hh = gg / (1.0 + jnp.exp(-gg)) * uu
        oo = lax.dot_general(hh, dw, (((1,), (1,)), ((), ())),
                             precision=_DOT_ALG, preferred_element_type=f32)
        o_ref[...] = _pack_rows(oo)


def _ffn_call(eot, act, par, nxt, xsort, gate_proj, up_proj, down_proj):
    grid_spec = pltpu.PrefetchScalarGridSpec(
        num_scalar_prefetch=4,
        grid=(NUM_TILES,),
        in_specs=[
            pl.BlockSpec((TILE, H // 2), lambda m, *_: (m, 0)),
            pl.BlockSpec(memory_space=pl.ANY),
            pl.BlockSpec(memory_space=pl.ANY),
            pl.BlockSpec(memory_space=pl.ANY),
        ],
        out_specs=pl.BlockSpec((TILE, H // 2), lambda m, *_: (m, 0)),
        scratch_shapes=[
            pltpu.VMEM((2, I, H), jnp.float32),
            pltpu.VMEM((2, I, H), jnp.float32),
            pltpu.VMEM((2, H, I), jnp.float32),
            pltpu.SemaphoreType.DMA((2,)),
        ],
    )
    return pl.pallas_call(
        _ffn_body,
        grid_spec=grid_spec,
        out_shape=jax.ShapeDtypeStruct((PADDED, H // 2), jnp.int32),
        compiler_params=pltpu.CompilerParams(
            dimension_semantics=("arbitrary",)),
    )(eot, act, par, nxt, xsort, gate_proj, up_proj, down_proj)


def _gather_body(os_hbm, ia_hbm, ib_hbm, ga_hbm, gb_hbm, ia_v, ib_v, abuf,
                 bbuf):
    wid = lax.axis_index("s") * 2 + lax.axis_index("c")
    pltpu.sync_copy(ia_hbm.at[wid], ia_v)
    pltpu.sync_copy(ib_hbm.at[wid], ib_v)
    for j in range(NSUB):
        base = wid * TOK_PER_W + j * SUB
        pltpu.sync_copy(os_hbm.at[ia_v.at[j]], abuf)
        pltpu.sync_copy(os_hbm.at[ib_v.at[j]], bbuf)
        pltpu.sync_copy(abuf, ga_hbm.at[pl.ds(base, SUB)])
        pltpu.sync_copy(bbuf, gb_hbm.at[pl.ds(base, SUB)])


def _gather_call(outsort, ia, ib):
    mesh = plsc.VectorSubcoreMesh(core_axis_name="c", subcore_axis_name="s")
    kern = pl.kernel(
        _gather_body,
        out_type=(jax.ShapeDtypeStruct((N, H // 2), jnp.int32),
                  jax.ShapeDtypeStruct((N, H // 2), jnp.int32)),
        mesh=mesh,
        scratch_types=[
            pltpu.VMEM((NSUB, SUB), jnp.int32),
            pltpu.VMEM((NSUB, SUB), jnp.int32),
            pltpu.VMEM((SUB, H // 2), jnp.int32),
            pltpu.VMEM((SUB, H // 2), jnp.int32),
        ],
    )
    return kern(outsort, ia, ib)


def _combine_body(ga_ref, gb_ref, wa_ref, wb_ref, y_ref):
    y_ref[...] = (wa_ref[...] * _unpack_rows(ga_ref[...])
                  + wb_ref[...] * _unpack_rows(gb_ref[...]))


def _combine_call(ga, gb, wa, wb):
    return pl.pallas_call(
        _combine_body,
        grid=(N // TILE,),
        in_specs=[
            pl.BlockSpec((TILE, H // 2), lambda m: (m, 0)),
            pl.BlockSpec((TILE, H // 2), lambda m: (m, 0)),
            pl.BlockSpec((TILE, 1), lambda m: (m, 0)),
            pl.BlockSpec((TILE, 1), lambda m: (m, 0)),
        ],
        out_specs=pl.BlockSpec((TILE, H), lambda m: (m, 0)),
        out_shape=jax.ShapeDtypeStruct((N, H), jnp.float32),
    )(ga, gb, wa, wb)


def kernel(hidden_states, gate_weight, e_bias, gate_proj, up_proj, down_proj):
    b, sq, h = hidden_states.shape
    x = hidden_states.reshape(N, H).astype(jnp.float32)
    eb = e_bias.reshape(NE, 1).astype(jnp.float32)

    (pos_a, pos_b, w_a, w_b, eot, act, par, nxt,
     xpk) = _routing_call(x, gate_weight, eb)
    ia_sc = pos_a.reshape(NWORK, NSUB, SUB)
    ib_sc = pos_b.reshape(NWORK, NSUB, SUB)

    xsort = _scatter_call(xpk, ia_sc, ib_sc)
    outsort = _ffn_call(eot[0, :NUM_TILES], act[0, :NUM_TILES],
                        par[0, :NUM_TILES], nxt[0, :NUM_TILES], xsort,
                        gate_proj, up_proj, down_proj)
    ga, gb = _gather_call(outsort, ia_sc, ib_sc)
    y = _combine_call(ga, gb, w_a.reshape(N, 1), w_b.reshape(N, 1))
    return y.reshape(b, sq, h)
